# baseline (speedup 1.0000x reference)
import functools

import jax
import jax.numpy as jnp
from jax import lax
from jax.experimental import pallas as pl
from jax.experimental.pallas import tpu as pltpu
from jax.experimental.pallas import tpu_sc as plsc


_GATHER_DNUMS = jax.lax.GatherDimensionNumbers(
    offset_dims=(), collapsed_slice_dims=(0,), start_index_map=(0,))


def _make_sc_logits(n_rows, d, base, n_sc_rows):
    mesh = plsc.VectorSubcoreMesh(core_axis_name="c", subcore_axis_name="s")
    nw = 32
    rows_pw = n_sc_rows // nw
    ch = 64
    nch = rows_pw // ch
    nj = d // 16

    @functools.partial(
        pl.kernel, mesh=mesh,
        out_type=jax.ShapeDtypeStruct((n_sc_rows,), jnp.float32),
        scratch_types=[
            pltpu.VMEM((d,), jnp.float32),
            pltpu.VMEM((ch, d), jnp.float32),
            pltpu.VMEM((ch, d), jnp.float32),
            pltpu.VMEM((ch,), jnp.float32),
            pltpu.SemaphoreType.DMA,
            pltpu.SemaphoreType.DMA,
        ],
    )
    def sc_logits(x_hbm, w_hbm, out_hbm, wbuf, xbuf0, xbuf1, lbuf, sem0, sem1):
        wid = lax.axis_index("s") * 2 + lax.axis_index("c")
        row0 = base + wid * rows_pw
        pltpu.sync_copy(w_hbm, wbuf)
        xbufs = (xbuf0, xbuf1)
        sems = (sem0, sem1)
        copies = [None, None]
        for c in range(min(2, nch)):
            copies[c] = pltpu.make_async_copy(
                x_hbm.at[pl.ds(row0 + c * ch, ch)], xbufs[c], sems[c])
            copies[c].start()
        lane = lax.iota(jnp.int32, 16)
        for c in range(nch):
            buf = xbufs[c % 2]
            copies[c % 2].wait()

            def group_body(g, _, buf=buf):
                def row_body(t, gres):
                    r = g * 16 + t
                    acc = jnp.zeros((16,), jnp.float32)
                    for j in range(nj):
                        acc = acc + (buf[r, pl.ds(j * 16, 16)]
                                     * wbuf[pl.ds(j * 16, 16)])
                    for sh in (8, 4, 2, 1):
                        acc = acc + lax.gather(
                            acc, (lane ^ sh)[:, None], _GATHER_DNUMS,
                            slice_sizes=(1,),
                            mode=lax.GatherScatterMode.PROMISE_IN_BOUNDS)
                    return jnp.where(lane == t, acc, gres)

                gres = lax.fori_loop(0, 16, row_body,
                                     jnp.zeros((16,), jnp.float32))
                lbuf[pl.ds(g * 16, 16)] = gres
                return 0

            lax.fori_loop(0, ch // 16, group_body, 0)
            if c + 2 < nch:
                copies[c % 2] = pltpu.make_async_copy(
                    x_hbm.at[pl.ds(row0 + (c + 2) * ch, ch)], xbufs[c % 2],
                    sems[c % 2])
                copies[c % 2].start()
            pltpu.sync_copy(lbuf, out_hbm.at[pl.ds(wid * rows_pw + c * ch, ch)])

    return sc_logits


def _logits_kernel(x_ref, w_ref, out_ref):
    lg = jax.lax.dot_general(
        w_ref[...], x_ref[...], (((1,), (1,)), ((), ())),
        preferred_element_type=jnp.float32)
    out_ref[...] = lg[None]


def _scale_kernel(logits_ref, scale_ref, *, k):
    l = logits_ref[0]
    nb, ns = l.shape
    u = jax.lax.bitcast_convert_type(l, jnp.int32)
    key = u ^ (jnp.int32(0x7FFFFFFF) & (u >> 31))

    lo = jnp.min(key, axis=1, keepdims=True)
    hi = jnp.max(key, axis=1, keepdims=True)

    def body(_, lh):
        lo, hi = lh
        xo = lo ^ hi
        mid = (lo & hi) + (xo >> 1) + (xo & 1)
        cnt = jnp.sum((key >= mid).astype(jnp.int32), axis=1, keepdims=True)
        ge = cnt >= k
        return jnp.where(ge, mid, lo), jnp.where(ge, hi, mid - 1)

    lo, hi = jax.lax.fori_loop(0, 34, body, (lo, hi))
    t = lo

    gt = key > t
    eq = key == t
    cnt_gt = jnp.sum(gt.astype(jnp.int32), axis=1, keepdims=True)
    r = k - cnt_gt
    iota = jax.lax.broadcasted_iota(jnp.int32, (nb, ns), 1)

    lo2 = jnp.ones_like(r)
    hi2 = jnp.full_like(r, ns)

    def body2(_, lh):
        lo, hi = lh
        mid = (lo + hi) >> 1
        cnt = jnp.sum((eq & (iota < mid)).astype(jnp.int32), axis=1,
                      keepdims=True)
        ge = cnt >= r
        return jnp.where(ge, lo, mid + 1), jnp.where(ge, mid, hi)

    lo2, _ = jax.lax.fori_loop(0, 14, body2, (lo2, hi2))
    selected = gt | (eq & (iota < lo2))

    m = jnp.max(l, axis=1, keepdims=True)
    e = jnp.exp(l - m)
    denom = jnp.sum(jnp.where(selected, e, 0.0), axis=1, keepdims=True)
    scale_ref[...] = (1.0 + jnp.where(selected, e / denom, 0.0))[:, None, :]


def _apply_kernel(x_ref, s_ref, out_ref):
    blk = x_ref.shape[1]
    s_col = jnp.reshape(s_ref[0], (blk, 1))
    out_ref[0] = x_ref[0] * s_col


def kernel(x, W_router):
    b, s, d = x.shape
    k = int(s * 0.5)
    blk = 4096
    wt = W_router.reshape(1, d)

    bs = b * s
    r_sc = 16384
    r_tc = bs - r_sc
    xf = x.reshape(bs, d)

    tc_logits = pl.pallas_call(
        _logits_kernel,
        grid=(r_tc // blk,),
        in_specs=[
            pl.BlockSpec((blk, d), lambda i: (i, 0)),
            pl.BlockSpec((1, d), lambda i: (0, 0)),
        ],
        out_specs=pl.BlockSpec((1, 1, blk), lambda i: (0, 0, i)),
        out_shape=jax.ShapeDtypeStruct((1, 1, r_tc), jnp.float32),
    )(xf, wt)

    sc_logits = _make_sc_logits(bs, d, r_tc, r_sc)(xf, W_router.reshape(d))

    logits = jnp.concatenate(
        [tc_logits.reshape(1, r_tc), sc_logits.reshape(1, r_sc)], axis=1)

    scale = pl.pallas_call(
        functools.partial(_scale_kernel, k=k),
        out_shape=jax.ShapeDtypeStruct((b, 1, s), jnp.float32),
    )(logits.reshape(1, b, s))

    out = pl.pallas_call(
        _apply_kernel,
        grid=(b, s // blk),
        in_specs=[
            pl.BlockSpec((1, blk, d), lambda i, j: (i, j, 0)),
            pl.BlockSpec((1, 1, blk), lambda i, j: (i, 0, j)),
        ],
        out_specs=pl.BlockSpec((1, blk, d), lambda i, j: (i, j, 0)),
        out_shape=jax.ShapeDtypeStruct((b, s, d), jnp.float32),
    )(x, scale)

    return out


# speedup vs baseline: 1.0425x; 1.0425x over previous
import functools

import jax
import jax.numpy as jnp
from jax import lax
from jax.experimental import pallas as pl
from jax.experimental.pallas import tpu as pltpu
from jax.experimental.pallas import tpu_sc as plsc


_GATHER_DNUMS = jax.lax.GatherDimensionNumbers(
    offset_dims=(), collapsed_slice_dims=(0,), start_index_map=(0,))


def _make_sc_logits(n_rows, d, base, n_sc_rows):
    mesh = plsc.VectorSubcoreMesh(core_axis_name="c", subcore_axis_name="s")
    nw = 32
    rows_pw = n_sc_rows // nw
    ch = 64
    nch = rows_pw // ch
    nj = d // 16

    @functools.partial(
        pl.kernel, mesh=mesh,
        out_type=jax.ShapeDtypeStruct((n_sc_rows,), jnp.float32),
        scratch_types=[
            pltpu.VMEM((d,), jnp.float32),
            pltpu.VMEM((ch, d), jnp.float32),
            pltpu.VMEM((ch, d), jnp.float32),
            pltpu.VMEM((ch,), jnp.float32),
            pltpu.SemaphoreType.DMA,
            pltpu.SemaphoreType.DMA,
        ],
    )
    def sc_logits(x_hbm, w_hbm, out_hbm, wbuf, xbuf0, xbuf1, lbuf, sem0, sem1):
        wid = lax.axis_index("s") * 2 + lax.axis_index("c")
        row0 = base + wid * rows_pw
        pltpu.sync_copy(w_hbm, wbuf)
        xbufs = (xbuf0, xbuf1)
        sems = (sem0, sem1)
        copies = [None, None]
        for c in range(min(2, nch)):
            copies[c] = pltpu.make_async_copy(
                x_hbm.at[pl.ds(row0 + c * ch, ch)], xbufs[c], sems[c])
            copies[c].start()
        lane = lax.iota(jnp.int32, 16)
        for c in range(nch):
            buf = xbufs[c % 2]
            copies[c % 2].wait()

            def group_body(g, _, buf=buf):
                def row_body(t, gres):
                    r = g * 16 + t
                    acc = jnp.zeros((16,), jnp.float32)
                    for j in range(nj):
                        acc = acc + (buf[r, pl.ds(j * 16, 16)]
                                     * wbuf[pl.ds(j * 16, 16)])
                    for sh in (8, 4, 2, 1):
                        acc = acc + lax.gather(
                            acc, (lane ^ sh)[:, None], _GATHER_DNUMS,
                            slice_sizes=(1,),
                            mode=lax.GatherScatterMode.PROMISE_IN_BOUNDS)
                    return jnp.where(lane == t, acc, gres)

                gres = lax.fori_loop(0, 16, row_body,
                                     jnp.zeros((16,), jnp.float32))
                lbuf[pl.ds(g * 16, 16)] = gres
                return 0

            lax.fori_loop(0, ch // 16, group_body, 0)
            if c + 2 < nch:
                copies[c % 2] = pltpu.make_async_copy(
                    x_hbm.at[pl.ds(row0 + (c + 2) * ch, ch)], xbufs[c % 2],
                    sems[c % 2])
                copies[c % 2].start()
            pltpu.sync_copy(lbuf, out_hbm.at[pl.ds(wid * rows_pw + c * ch, ch)])

    return sc_logits


def _logits_kernel(x_ref, w_ref, out_ref):
    lg = jax.lax.dot_general(
        w_ref[...], x_ref[...], (((1,), (1,)), ((), ())),
        preferred_element_type=jnp.float32)
    out_ref[...] = lg[None]


def _scale_kernel(logits_ref, scale_ref, *, k):
    l = logits_ref[0]
    nb, ns = l.shape
    u = jax.lax.bitcast_convert_type(l, jnp.int32)
    key = u ^ (jnp.int32(0x7FFFFFFF) & (u >> 31))

    lo = jnp.min(key, axis=1, keepdims=True)
    hi = jnp.max(key, axis=1, keepdims=True)

    def body(_, lh):
        lo, hi = lh
        xo = lo ^ hi
        mid = (lo & hi) + (xo >> 1) + (xo & 1)
        cnt = jnp.sum((key >= mid).astype(jnp.int32), axis=1, keepdims=True)
        ge = cnt >= k
        return jnp.where(ge, mid, lo), jnp.where(ge, hi, mid - 1)

    lo, hi = jax.lax.fori_loop(0, 34, body, (lo, hi))
    t = lo

    gt = key > t
    eq = key == t
    cnt_gt = jnp.sum(gt.astype(jnp.int32), axis=1, keepdims=True)
    r = k - cnt_gt
    iota = jax.lax.broadcasted_iota(jnp.int32, (nb, ns), 1)

    lo2 = jnp.ones_like(r)
    hi2 = jnp.full_like(r, ns)

    def body2(_, lh):
        lo, hi = lh
        mid = (lo + hi) >> 1
        cnt = jnp.sum((eq & (iota < mid)).astype(jnp.int32), axis=1,
                      keepdims=True)
        ge = cnt >= r
        return jnp.where(ge, lo, mid + 1), jnp.where(ge, mid, hi)

    lo2, _ = jax.lax.fori_loop(0, 14, body2, (lo2, hi2))
    selected = gt | (eq & (iota < lo2))

    m = jnp.max(l, axis=1, keepdims=True)
    e = jnp.exp(l - m)
    denom = jnp.sum(jnp.where(selected, e, 0.0), axis=1, keepdims=True)
    scale_ref[...] = (1.0 + jnp.where(selected, e / denom, 0.0))[:, None, :]


def _apply_kernel(x_ref, s_ref, out_ref):
    blk = x_ref.shape[1]
    s_col = jnp.reshape(s_ref[0], (blk, 1))
    out_ref[0] = x_ref[0] * s_col


def kernel(x, W_router):
    b, s, d = x.shape
    k = int(s * 0.5)
    blk = 4096
    wt = W_router.reshape(1, d)

    bs = b * s
    r_sc = 8192
    r_tc = bs - r_sc
    xf = x.reshape(bs, d)

    tc_logits = pl.pallas_call(
        _logits_kernel,
        grid=(r_tc // blk,),
        in_specs=[
            pl.BlockSpec((blk, d), lambda i: (i, 0)),
            pl.BlockSpec((1, d), lambda i: (0, 0)),
        ],
        out_specs=pl.BlockSpec((1, 1, blk), lambda i: (0, 0, i)),
        out_shape=jax.ShapeDtypeStruct((1, 1, r_tc), jnp.float32),
    )(xf, wt)

    sc_logits = _make_sc_logits(bs, d, r_tc, r_sc)(xf, W_router.reshape(d))

    logits = jnp.concatenate(
        [tc_logits.reshape(1, r_tc), sc_logits.reshape(1, r_sc)], axis=1)

    scale = pl.pallas_call(
        functools.partial(_scale_kernel, k=k),
        out_shape=jax.ShapeDtypeStruct((b, 1, s), jnp.float32),
    )(logits.reshape(1, b, s))

    out = pl.pallas_call(
        _apply_kernel,
        grid=(b, s // blk),
        in_specs=[
            pl.BlockSpec((1, blk, d), lambda i, j: (i, j, 0)),
            pl.BlockSpec((1, 1, blk), lambda i, j: (i, 0, j)),
        ],
        out_specs=pl.BlockSpec((1, blk, d), lambda i, j: (i, j, 0)),
        out_shape=jax.ShapeDtypeStruct((b, s, d), jnp.float32),
    )(x, scale)

    return out


# speedup vs baseline: 1.3283x; 1.2742x over previous
import functools

import jax
import jax.numpy as jnp
from jax import lax
from jax.experimental import pallas as pl
from jax.experimental.pallas import tpu as pltpu


def _fused_kernel(x_ref, w_ref, out_ref, lscr, sscr, *, k, blk, split, bpb):
    i = pl.program_id(0)

    @pl.when(i < split)
    def _logits():
        lg = lax.dot_general(
            w_ref[...], x_ref[...], (((1,), (1,)), ((), ())),
            preferred_element_type=jnp.float32)
        lscr[i // bpb, pl.ds((i % bpb) * blk, blk)] = lg.reshape(blk)

    @pl.when(i == split)
    def _route():
        l = lscr[...]
        nb, ns = l.shape
        u = lax.bitcast_convert_type(l, jnp.int32)
        key = u ^ (jnp.int32(0x7FFFFFFF) & (u >> 31))

        lo = jnp.min(key, axis=1, keepdims=True)
        hi = jnp.max(key, axis=1, keepdims=True)

        def body(_, lh):
            lo, hi = lh
            xo = lo ^ hi
            mid = (lo & hi) + (xo >> 1) + (xo & 1)
            cnt = jnp.sum((key >= mid).astype(jnp.int32), axis=1,
                          keepdims=True)
            ge = cnt >= k
            return jnp.where(ge, mid, lo), jnp.where(ge, hi, mid - 1)

        lo, hi = lax.fori_loop(0, 34, body, (lo, hi))
        t = lo

        gt = key > t
        eq = key == t
        cnt_gt = jnp.sum(gt.astype(jnp.int32), axis=1, keepdims=True)
        r = k - cnt_gt
        iota = lax.broadcasted_iota(jnp.int32, (nb, ns), 1)

        lo2 = jnp.ones_like(r)
        hi2 = jnp.full_like(r, ns)

        def body2(_, lh):
            lo, hi = lh
            mid = (lo + hi) >> 1
            cnt = jnp.sum((eq & (iota < mid)).astype(jnp.int32), axis=1,
                          keepdims=True)
            ge = cnt >= r
            return jnp.where(ge, lo, mid + 1), jnp.where(ge, mid, hi)

        lo2, _ = lax.fori_loop(0, 14, body2, (lo2, hi2))
        selected = gt | (eq & (iota < lo2))

        m = jnp.max(l, axis=1, keepdims=True)
        e = jnp.exp(l - m)
        denom = jnp.sum(jnp.where(selected, e, 0.0), axis=1, keepdims=True)
        sscr[...] = 1.0 + jnp.where(selected, e / denom, 0.0)

    @pl.when(i >= split)
    def _apply():
        j = i - split
        s_col = jnp.reshape(sscr[j // bpb, pl.ds((j % bpb) * blk, blk)],
                            (blk, 1))
        out_ref[...] = x_ref[...] * s_col


def kernel(x, W_router):
    b, s, d = x.shape
    k = int(s * 0.5)
    blk = 4096
    bs = b * s
    split = bs // blk
    bpb = s // blk
    xf = x.reshape(bs, d)
    wt = W_router.reshape(1, d)

    out = pl.pallas_call(
        functools.partial(_fused_kernel, k=k, blk=blk, split=split, bpb=bpb),
        grid=(2 * split,),
        in_specs=[
            pl.BlockSpec((blk, d), lambda i, split=split: (i % split, 0)),
            pl.BlockSpec((1, d), lambda i: (0, 0)),
        ],
        out_specs=pl.BlockSpec(
            (blk, d), lambda i, split=split: (jnp.maximum(i - split, 0), 0)),
        out_shape=jax.ShapeDtypeStruct((bs, d), jnp.float32),
        scratch_shapes=[
            pltpu.VMEM((b, s), jnp.float32),
            pltpu.VMEM((b, s), jnp.float32),
        ],
    )(xf, wt)

    return out.reshape(b, s, d)
